# baseline (speedup 1.0000x reference)
import functools
import jax
import jax.numpy as jnp
from jax import lax
from jax.experimental import pallas as pl
from jax.experimental.pallas import tpu as pltpu
from jax.experimental.pallas import tpu_sc as plsc

_N = 2048
_D = 1024
_F = 2048
_E = 8
_K = 2
_A = _N * _K
_BLK = 256
_NBLK = 24
_LPAD = _NBLK * _BLK

_NC, _NS, _L = 2, 16, 16
_NW = _NC * _NS
_APW = _A // _NW
_TPW = _N // _NW


def _route_body(x_ref, wg_ref, bg_ref,
                pos_ref, wkb_ref, be_ref, lb_ref, xp_ref):
    xs = x_ref[...]
    logits = jnp.dot(xs, wg_ref[...], preferred_element_type=jnp.float32)
    logits = logits + bg_ref[...]
    m = jnp.max(logits, axis=1, keepdims=True)
    ex = jnp.exp(logits - m)
    probs = ex / jnp.sum(ex, axis=1, keepdims=True)

    lane = lax.broadcasted_iota(jnp.int32, (_N, _E), 1)
    m1 = jnp.max(probs, axis=1, keepdims=True)
    i1 = jnp.min(jnp.where(probs == m1, lane, _E), axis=1, keepdims=True)
    masked = jnp.where(lane == i1, -jnp.inf, probs)
    m2 = jnp.max(masked, axis=1, keepdims=True)
    i2 = jnp.min(jnp.where(masked == m2, lane, _E), axis=1, keepdims=True)
    denom = m1 + m2

    ea = jnp.concatenate([i1, i2], axis=0)
    wk = jnp.concatenate([m1 / denom, m2 / denom], axis=0)
    wkb_ref[...] = jnp.broadcast_to(wk, (_A, _L))

    laneA = lax.broadcasted_iota(jnp.int32, (_A, _E), 1)
    oh = (ea == laneA).astype(jnp.float32)
    counts = jnp.sum(oh, axis=0, keepdims=True)
    cpad = jnp.floor((counts + (_BLK - 1)) * (1.0 / _BLK)) * _BLK

    r8 = lax.broadcasted_iota(jnp.int32, (_E, _E), 0)
    c8 = lax.broadcasted_iota(jnp.int32, (_E, _E), 1)
    offp = jnp.dot(cpad, (r8 < c8).astype(jnp.float32),
                   preferred_element_type=jnp.float32)

    ch = 512
    r5 = lax.broadcasted_iota(jnp.int32, (ch, ch), 0)
    c5 = lax.broadcasted_iota(jnp.int32, (ch, ch), 1)
    ltri = (r5 > c5).astype(jnp.float32)
    carry = jnp.zeros((1, _E), jnp.float32)
    ranks = []
    for i in range(_A // ch):
        ohc = oh[i * ch:(i + 1) * ch]
        ranks.append(jnp.dot(ltri, ohc,
                             preferred_element_type=jnp.float32) + carry)
        carry = carry + jnp.sum(ohc, axis=0, keepdims=True)
    rw = jnp.concatenate(ranks, axis=0)

    posf = jnp.sum(oh * (offp + rw), axis=1, keepdims=True)
    pos_ref[...] = posf.astype(jnp.int32).reshape(_A // 128, 1, 128)

    sb = lax.broadcasted_iota(jnp.int32, (_NBLK, _E), 0).astype(jnp.float32) * _BLK
    eidx = lax.broadcasted_iota(jnp.int32, (_NBLK, _E), 1).astype(jnp.float32)
    active = (sb >= offp) & (sb < offp + cpad)
    beval = jnp.sum(jnp.where(active, eidx, 0.0), axis=1, keepdims=True)
    total = jnp.sum(cpad)
    beval = jnp.where(sb[:, 0:1] >= total, float(_E - 1), beval)
    be_ref[...] = beval.astype(jnp.int32).reshape(1, _NBLK)

    colmean = jnp.mean(probs, axis=0, keepdims=True)
    mu = jnp.mean(colmean)
    lb_ref[...] = (jnp.sum((colmean - mu) ** 2) / (_E - 1)).reshape(1, 1)

    u = lax.bitcast_convert_type(xs, jnp.uint32)
    rb = (u + jnp.uint32(0x7FFF) + ((u >> 16) & jnp.uint32(1))) >> 16
    xp_ref[...] = rb[:, :_D // 2] | (rb[:, _D // 2:] << 16)


def _route(xs, Wg, bg):
    return pl.pallas_call(
        _route_body,
        in_specs=[
            pl.BlockSpec((_N, _D), lambda: (0, 0)),
            pl.BlockSpec((_D, _E), lambda: (0, 0)),
            pl.BlockSpec((1, _E), lambda: (0, 0)),
        ],
        out_specs=[
            pl.BlockSpec((_A // 128, 1, 128), lambda: (0, 0, 0)),
            pl.BlockSpec((_A, _L), lambda: (0, 0)),
            pl.BlockSpec((1, _NBLK), lambda: (0, 0)),
            pl.BlockSpec((1, 1), lambda: (0, 0)),
            pl.BlockSpec((_N, _D // 2), lambda: (0, 0)),
        ],
        out_shape=[
            jax.ShapeDtypeStruct((_A // 128, 1, 128), jnp.int32),
            jax.ShapeDtypeStruct((_A, _L), jnp.float32),
            jax.ShapeDtypeStruct((1, _NBLK), jnp.int32),
            jax.ShapeDtypeStruct((1, 1), jnp.float32),
            jax.ShapeDtypeStruct((_N, _D // 2), jnp.uint32),
        ],
    )(xs, Wg, bg.reshape(1, _E))


def _dispatch(pos, xs):
    mesh = plsc.VectorSubcoreMesh(core_axis_name="c", subcore_axis_name="s")
    rpw = _LPAD // _NW
    ch = 32
    nch = rpw // ch

    nca = _APW // ch

    @functools.partial(
        pl.kernel, mesh=mesh,
        out_type=jax.ShapeDtypeStruct((_LPAD, _D // 2), jnp.uint32),
        scratch_types=(
            [pltpu.VMEM((ch,), jnp.int32) for _ in range(4)]
            + [pltpu.VMEM((ch,), jnp.int32) for _ in range(4)]
            + [pltpu.VMEM((128,), jnp.int32)]
            + [pltpu.VMEM((ch, _D // 2), jnp.uint32) for _ in range(2)]
            + [pltpu.SemaphoreType.DMA, pltpu.SemaphoreType.DMA,
               pltpu.SemaphoreType.DMA]
        ),
    )
    def k(pos_hbm, x_hbm, xs_hbm,
          tok0, tok1, tok2, tok3, pos0, pos1, pos2, pos3, pw, rows0, rows1,
          semp, semg, sems):
        wid = lax.axis_index("s") * _NC + lax.axis_index("c")
        base = wid * _APW
        toks = (tok0, tok1, tok2, tok3)
        poss = (pos0, pos1, pos2, pos3)
        rows = (rows0, rows1)
        i16 = lax.broadcasted_iota(jnp.int32, (_L,), 0)
        for c in range(nca):
            for t in range(ch // _L):
                toks[c][pl.ds(t * _L, _L)] = (
                    (base + c * ch + t * _L + i16) & (_N - 1))
        pltpu.sync_copy(pos_hbm.at[wid, 0], pw)
        g = [pltpu.async_copy(x_hbm.at[toks[c]], rows[c], semg)
             for c in range(2)]
        for c in range(nca):
            for t in range(ch // _L):
                poss[c][pl.ds(t * _L, _L)] = pw[pl.ds(c * ch + t * _L, _L)]
        s_prev = None
        for c in range(nca):
            b = c % 2
            g[b].wait()
            s = pltpu.async_copy(rows[b], xs_hbm.at[poss[c]], sems)
            if s_prev is not None:
                s_prev.wait()
            if c + 2 < nca:
                s.wait()
                g[b] = pltpu.async_copy(x_hbm.at[toks[c + 2]], rows[b], semg)
                s_prev = None
            else:
                s_prev = s
        s_prev.wait()

    return k(pos, xs)


def _gmm_body(be_sref, xs_ref, w1_ref, b1_ref, w2_ref, b2_ref, ys_ref,
              w1b_ref, w2b_ref):
    i = pl.program_id(0)
    prev = be_sref[jnp.maximum(i - 1, 0)]

    @pl.when((i == 0) | (be_sref[i] != prev))
    def _cast():
        w1b_ref[...] = w1_ref[0].astype(jnp.bfloat16)
        w2b_ref[...] = w2_ref[0].astype(jnp.bfloat16)

    xi = xs_ref[...]
    lo_f = lax.bitcast_convert_type(xi << 16, jnp.float32)
    hi_f = lax.bitcast_convert_type(xi & jnp.uint32(0xFFFF0000), jnp.float32)
    xb = jnp.concatenate([lo_f, hi_f], axis=1).astype(jnp.bfloat16)
    h = jnp.dot(xb, w1b_ref[...], preferred_element_type=jnp.float32)
    h = jnp.maximum(h + b1_ref[0], 0.0)
    y = jnp.dot(h.astype(jnp.bfloat16), w2b_ref[...],
                preferred_element_type=jnp.float32)
    ys_ref[...] = y + b2_ref[0]


def _gmm(be, xs_sorted, W1, b1, W2, b2):
    grid_spec = pltpu.PrefetchScalarGridSpec(
        num_scalar_prefetch=1,
        grid=(_NBLK,),
        in_specs=[
            pl.BlockSpec((_BLK, _D // 2), lambda i, be: (i, 0)),
            pl.BlockSpec((1, _D, _F), lambda i, be: (be[i], 0, 0)),
            pl.BlockSpec((1, 1, _F), lambda i, be: (be[i], 0, 0)),
            pl.BlockSpec((1, _F, _D), lambda i, be: (be[i], 0, 0)),
            pl.BlockSpec((1, 1, _D), lambda i, be: (be[i], 0, 0)),
        ],
        out_specs=pl.BlockSpec((_BLK, _D), lambda i, be: (i, 0)),
        scratch_shapes=[
            pltpu.VMEM((_D, _F), jnp.bfloat16),
            pltpu.VMEM((_F, _D), jnp.bfloat16),
        ],
    )
    return pl.pallas_call(
        _gmm_body,
        grid_spec=grid_spec,
        out_shape=jax.ShapeDtypeStruct((_LPAD, _D), jnp.float32),
    )(be, xs_sorted, W1, b1.reshape(_E, 1, _F), W2, b2.reshape(_E, 1, _D))


def _combine(pos, wkb, ys):
    mesh = plsc.VectorSubcoreMesh(core_axis_name="c", subcore_axis_name="s")
    ch = 32

    @functools.partial(
        pl.kernel, mesh=mesh,
        out_type=jax.ShapeDtypeStruct((_N, _D), jnp.float32),
        scratch_types=[
            pltpu.VMEM((ch,), jnp.int32),
            pltpu.VMEM((ch,), jnp.int32),
            pltpu.VMEM((128,), jnp.int32),
            pltpu.VMEM((128,), jnp.int32),
            pltpu.VMEM((ch, _L), jnp.float32),
            pltpu.VMEM((ch, _L), jnp.float32),
            pltpu.VMEM((ch, _D), jnp.float32),
            pltpu.VMEM((ch, _D), jnp.float32),
            pltpu.VMEM((ch, _D), jnp.float32),
            pltpu.SemaphoreType.DMA,
        ],
    )
    def k(pos_hbm, wkb_hbm, ys_hbm, out_hbm,
          posa_v, posb_v, pwa_v, pwb_v, wka_v, wkb_v,
          rowsa_v, rowsb_v, out_v, sem):
        wid = lax.axis_index("s") * _NC + lax.axis_index("c")
        tbase = wid * _TPW
        pltpu.sync_copy(pos_hbm.at[wid >> 1, 0], pwa_v)
        pltpu.sync_copy(pos_hbm.at[(_N // 128) + (wid >> 1), 0], pwb_v)
        col0 = (wid & 1) * _TPW
        for c in range(_TPW // ch):
            off = tbase + c * ch
            for t in range(ch // _L):
                s = pl.ds(t * _L, _L)
                posa_v[s] = pwa_v[pl.ds(col0 + c * ch + t * _L, _L)]
                posb_v[s] = pwb_v[pl.ds(col0 + c * ch + t * _L, _L)]
            pltpu.sync_copy(wkb_hbm.at[pl.ds(off, ch)], wka_v)
            pltpu.sync_copy(wkb_hbm.at[pl.ds(_N + off, ch)], wkb_v)
            pltpu.async_copy(ys_hbm.at[posa_v], rowsa_v, sem).wait()
            pltpu.async_copy(ys_hbm.at[posb_v], rowsb_v, sem).wait()

            def body(q, _):
                s = pl.ds(q * _L, _L)
                for j in range(ch):
                    out_v[j, s] = (wka_v[j] * rowsa_v[j, s]
                                   + wkb_v[j] * rowsb_v[j, s])
                return _

            lax.fori_loop(0, _D // _L, body, 0)
            pltpu.sync_copy(out_v, out_hbm.at[pl.ds(off, ch)])

    return k(pos, wkb, ys)


@jax.jit
def kernel(x, Wg, bg, W1, b1, W2, b2):
    xs = x.reshape(_N, _D)
    pos, wkb, be, lb, xp = _route(xs, Wg, bg)
    xs_sorted = _dispatch(pos, xp)
    ys = _gmm(be.reshape(_NBLK), xs_sorted, W1, b1, W2, b2)
    out = _combine(pos, wkb, ys)
    return out.reshape(x.shape), lb[0, 0]


# speedup vs baseline: 1.0536x; 1.0536x over previous
import functools
import jax
import jax.numpy as jnp
from jax import lax
from jax.experimental import pallas as pl
from jax.experimental.pallas import tpu as pltpu
from jax.experimental.pallas import tpu_sc as plsc

_N = 2048
_D = 1024
_F = 2048
_E = 8
_K = 2
_A = _N * _K
_BLK = 512
_NBLK = 16
_LPAD = _NBLK * _BLK

_NC, _NS, _L = 2, 16, 16
_NW = _NC * _NS
_APW = _A // _NW
_TPW = _N // _NW


def _route_body(x_ref, wg_ref, bg_ref,
                pos_ref, wkb_ref, be_ref, lb_ref, xp_ref):
    xs = x_ref[...]
    logits = jnp.dot(xs, wg_ref[...], preferred_element_type=jnp.float32)
    logits = logits + bg_ref[...]
    m = jnp.max(logits, axis=1, keepdims=True)
    ex = jnp.exp(logits - m)
    probs = ex / jnp.sum(ex, axis=1, keepdims=True)

    lane = lax.broadcasted_iota(jnp.int32, (_N, _E), 1)
    m1 = jnp.max(probs, axis=1, keepdims=True)
    i1 = jnp.min(jnp.where(probs == m1, lane, _E), axis=1, keepdims=True)
    masked = jnp.where(lane == i1, -jnp.inf, probs)
    m2 = jnp.max(masked, axis=1, keepdims=True)
    i2 = jnp.min(jnp.where(masked == m2, lane, _E), axis=1, keepdims=True)
    denom = m1 + m2

    ea = jnp.concatenate([i1, i2], axis=0)
    wk = jnp.concatenate([m1 / denom, m2 / denom], axis=0)
    wkb_ref[...] = jnp.broadcast_to(wk, (_A, _L))

    laneA = lax.broadcasted_iota(jnp.int32, (_A, _E), 1)
    oh = (ea == laneA).astype(jnp.float32)
    counts = jnp.sum(oh, axis=0, keepdims=True)
    cpad = jnp.floor((counts + (_BLK - 1)) * (1.0 / _BLK)) * _BLK

    r8 = lax.broadcasted_iota(jnp.int32, (_E, _E), 0)
    c8 = lax.broadcasted_iota(jnp.int32, (_E, _E), 1)
    offp = jnp.dot(cpad, (r8 < c8).astype(jnp.float32),
                   preferred_element_type=jnp.float32)

    ch = 512
    r5 = lax.broadcasted_iota(jnp.int32, (ch, ch), 0)
    c5 = lax.broadcasted_iota(jnp.int32, (ch, ch), 1)
    ltri = (r5 > c5).astype(jnp.float32)
    carry = jnp.zeros((1, _E), jnp.float32)
    ranks = []
    for i in range(_A // ch):
        ohc = oh[i * ch:(i + 1) * ch]
        ranks.append(jnp.dot(ltri, ohc,
                             preferred_element_type=jnp.float32) + carry)
        carry = carry + jnp.sum(ohc, axis=0, keepdims=True)
    rw = jnp.concatenate(ranks, axis=0)

    posf = jnp.sum(oh * (offp + rw), axis=1, keepdims=True)
    pos_ref[...] = posf.astype(jnp.int32).reshape(_A // 128, 1, 128)

    sb = lax.broadcasted_iota(jnp.int32, (_NBLK, _E), 0).astype(jnp.float32) * _BLK
    eidx = lax.broadcasted_iota(jnp.int32, (_NBLK, _E), 1).astype(jnp.float32)
    active = (sb >= offp) & (sb < offp + cpad)
    beval = jnp.sum(jnp.where(active, eidx, 0.0), axis=1, keepdims=True)
    total = jnp.sum(cpad)
    beval = jnp.where(sb[:, 0:1] >= total, float(_E - 1), beval)
    be_ref[...] = beval.astype(jnp.int32).reshape(1, _NBLK)

    colmean = jnp.mean(probs, axis=0, keepdims=True)
    mu = jnp.mean(colmean)
    lb_ref[...] = (jnp.sum((colmean - mu) ** 2) / (_E - 1)).reshape(1, 1)

    u = lax.bitcast_convert_type(xs, jnp.uint32)
    rb = (u + jnp.uint32(0x7FFF) + ((u >> 16) & jnp.uint32(1))) >> 16
    xp_ref[...] = rb[:, :_D // 2] | (rb[:, _D // 2:] << 16)


def _route(xs, Wg, bg):
    return pl.pallas_call(
        _route_body,
        in_specs=[
            pl.BlockSpec((_N, _D), lambda: (0, 0)),
            pl.BlockSpec((_D, _E), lambda: (0, 0)),
            pl.BlockSpec((1, _E), lambda: (0, 0)),
        ],
        out_specs=[
            pl.BlockSpec((_A // 128, 1, 128), lambda: (0, 0, 0)),
            pl.BlockSpec((_A, _L), lambda: (0, 0)),
            pl.BlockSpec((1, _NBLK), lambda: (0, 0)),
            pl.BlockSpec((1, 1), lambda: (0, 0)),
            pl.BlockSpec((_N, _D // 2), lambda: (0, 0)),
        ],
        out_shape=[
            jax.ShapeDtypeStruct((_A // 128, 1, 128), jnp.int32),
            jax.ShapeDtypeStruct((_A, _L), jnp.float32),
            jax.ShapeDtypeStruct((1, _NBLK), jnp.int32),
            jax.ShapeDtypeStruct((1, 1), jnp.float32),
            jax.ShapeDtypeStruct((_N, _D // 2), jnp.uint32),
        ],
    )(xs, Wg, bg.reshape(1, _E))


def _dispatch(pos, xs):
    mesh = plsc.VectorSubcoreMesh(core_axis_name="c", subcore_axis_name="s")
    rpw = _LPAD // _NW
    ch = 32
    nch = rpw // ch

    nca = _APW // ch

    @functools.partial(
        pl.kernel, mesh=mesh,
        out_type=jax.ShapeDtypeStruct((_LPAD, _D // 2), jnp.uint32),
        scratch_types=(
            [pltpu.VMEM((ch,), jnp.int32) for _ in range(4)]
            + [pltpu.VMEM((ch,), jnp.int32) for _ in range(4)]
            + [pltpu.VMEM((128,), jnp.int32)]
            + [pltpu.VMEM((ch, _D // 2), jnp.uint32) for _ in range(2)]
            + [pltpu.SemaphoreType.DMA, pltpu.SemaphoreType.DMA,
               pltpu.SemaphoreType.DMA]
        ),
    )
    def k(pos_hbm, x_hbm, xs_hbm,
          tok0, tok1, tok2, tok3, pos0, pos1, pos2, pos3, pw, rows0, rows1,
          semp, semg, sems):
        wid = lax.axis_index("s") * _NC + lax.axis_index("c")
        base = wid * _APW
        toks = (tok0, tok1, tok2, tok3)
        poss = (pos0, pos1, pos2, pos3)
        rows = (rows0, rows1)
        i16 = lax.broadcasted_iota(jnp.int32, (_L,), 0)
        for c in range(nca):
            for t in range(ch // _L):
                toks[c][pl.ds(t * _L, _L)] = (
                    (base + c * ch + t * _L + i16) & (_N - 1))
        pltpu.sync_copy(pos_hbm.at[wid, 0], pw)
        g = [pltpu.async_copy(x_hbm.at[toks[c]], rows[c], semg)
             for c in range(2)]
        for c in range(nca):
            for t in range(ch // _L):
                poss[c][pl.ds(t * _L, _L)] = pw[pl.ds(c * ch + t * _L, _L)]
        s_prev = None
        for c in range(nca):
            b = c % 2
            g[b].wait()
            s = pltpu.async_copy(rows[b], xs_hbm.at[poss[c]], sems)
            if s_prev is not None:
                s_prev.wait()
            if c + 2 < nca:
                s.wait()
                g[b] = pltpu.async_copy(x_hbm.at[toks[c + 2]], rows[b], semg)
                s_prev = None
            else:
                s_prev = s
        s_prev.wait()

    return k(pos, xs)


def _gmm_body(be_sref, xs_ref, w1_ref, b1_ref, w2_ref, b2_ref, ys_ref,
              w1b_ref, w2b_ref):
    i = pl.program_id(0)
    prev = be_sref[jnp.maximum(i - 1, 0)]

    @pl.when((i == 0) | (be_sref[i] != prev))
    def _cast():
        w1b_ref[...] = w1_ref[0].astype(jnp.bfloat16)
        w2b_ref[...] = w2_ref[0].astype(jnp.bfloat16)

    xi = xs_ref[...]
    lo_f = lax.bitcast_convert_type(xi << 16, jnp.float32)
    hi_f = lax.bitcast_convert_type(xi & jnp.uint32(0xFFFF0000), jnp.float32)
    xb = jnp.concatenate([lo_f, hi_f], axis=1).astype(jnp.bfloat16)
    h = jnp.dot(xb, w1b_ref[...], preferred_element_type=jnp.float32)
    h = jnp.maximum(h + b1_ref[0], 0.0)
    y = jnp.dot(h.astype(jnp.bfloat16), w2b_ref[...],
                preferred_element_type=jnp.float32)
    ys_ref[...] = y + b2_ref[0]


def _gmm(be, xs_sorted, W1, b1, W2, b2):
    grid_spec = pltpu.PrefetchScalarGridSpec(
        num_scalar_prefetch=1,
        grid=(_NBLK,),
        in_specs=[
            pl.BlockSpec((_BLK, _D // 2), lambda i, be: (i, 0)),
            pl.BlockSpec((1, _D, _F), lambda i, be: (be[i], 0, 0)),
            pl.BlockSpec((1, 1, _F), lambda i, be: (be[i], 0, 0)),
            pl.BlockSpec((1, _F, _D), lambda i, be: (be[i], 0, 0)),
            pl.BlockSpec((1, 1, _D), lambda i, be: (be[i], 0, 0)),
        ],
        out_specs=pl.BlockSpec((_BLK, _D), lambda i, be: (i, 0)),
        scratch_shapes=[
            pltpu.VMEM((_D, _F), jnp.bfloat16),
            pltpu.VMEM((_F, _D), jnp.bfloat16),
        ],
    )
    return pl.pallas_call(
        _gmm_body,
        grid_spec=grid_spec,
        out_shape=jax.ShapeDtypeStruct((_LPAD, _D), jnp.float32),
    )(be, xs_sorted, W1, b1.reshape(_E, 1, _F), W2, b2.reshape(_E, 1, _D))


def _combine(pos, wkb, ys):
    mesh = plsc.VectorSubcoreMesh(core_axis_name="c", subcore_axis_name="s")
    ch = 32

    @functools.partial(
        pl.kernel, mesh=mesh,
        out_type=jax.ShapeDtypeStruct((_N, _D), jnp.float32),
        scratch_types=[
            pltpu.VMEM((ch,), jnp.int32),
            pltpu.VMEM((ch,), jnp.int32),
            pltpu.VMEM((128,), jnp.int32),
            pltpu.VMEM((128,), jnp.int32),
            pltpu.VMEM((ch, _L), jnp.float32),
            pltpu.VMEM((ch, _L), jnp.float32),
            pltpu.VMEM((ch, _D), jnp.float32),
            pltpu.VMEM((ch, _D), jnp.float32),
            pltpu.VMEM((ch, _D), jnp.float32),
            pltpu.SemaphoreType.DMA,
        ],
    )
    def k(pos_hbm, wkb_hbm, ys_hbm, out_hbm,
          posa_v, posb_v, pwa_v, pwb_v, wka_v, wkb_v,
          rowsa_v, rowsb_v, out_v, sem):
        wid = lax.axis_index("s") * _NC + lax.axis_index("c")
        tbase = wid * _TPW
        pltpu.sync_copy(pos_hbm.at[wid >> 1, 0], pwa_v)
        pltpu.sync_copy(pos_hbm.at[(_N // 128) + (wid >> 1), 0], pwb_v)
        col0 = (wid & 1) * _TPW
        for c in range(_TPW // ch):
            off = tbase + c * ch
            for t in range(ch // _L):
                s = pl.ds(t * _L, _L)
                posa_v[s] = pwa_v[pl.ds(col0 + c * ch + t * _L, _L)]
                posb_v[s] = pwb_v[pl.ds(col0 + c * ch + t * _L, _L)]
            pltpu.sync_copy(wkb_hbm.at[pl.ds(off, ch)], wka_v)
            pltpu.sync_copy(wkb_hbm.at[pl.ds(_N + off, ch)], wkb_v)
            pltpu.async_copy(ys_hbm.at[posa_v], rowsa_v, sem).wait()
            pltpu.async_copy(ys_hbm.at[posb_v], rowsb_v, sem).wait()

            def body(q, _):
                s = pl.ds(q * _L, _L)
                for j in range(ch):
                    out_v[j, s] = (wka_v[j] * rowsa_v[j, s]
                                   + wkb_v[j] * rowsb_v[j, s])
                return _

            lax.fori_loop(0, _D // _L, body, 0)
            pltpu.sync_copy(out_v, out_hbm.at[pl.ds(off, ch)])

    return k(pos, wkb, ys)


@jax.jit
def kernel(x, Wg, bg, W1, b1, W2, b2):
    xs = x.reshape(_N, _D)
    pos, wkb, be, lb, xp = _route(xs, Wg, bg)
    xs_sorted = _dispatch(pos, xp)
    ys = _gmm(be.reshape(_NBLK), xs_sorted, W1, b1, W2, b2)
    out = _combine(pos, wkb, ys)
    return out.reshape(x.shape), lb[0, 0]
